# Initial kernel scaffold; baseline (speedup 1.0000x reference)
import functools

import jax
import jax.numpy as jnp
from jax import lax
from jax.experimental import pallas as pl
from jax.experimental.pallas import tpu as pltpu
from jax.experimental.pallas import tpu_sc as plsc

NC = 2
NS = 16
LANES = 16
CH = 128
NB = 8


def _rup(x, m):
    return (x + m - 1) // m * m


def _mesh():
    return plsc.VectorSubcoreMesh(
        core_axis_name="c", subcore_axis_name="s", num_cores=NC, num_subcores=NS
    )


def _build_prep(ep, n_nodes, half, np_pad):
    units = ep // (NC * NS * CH)
    deg_stripe = np_pad // NS

    @functools.partial(
        pl.kernel,
        out_type=(
            jax.ShapeDtypeStruct((NC, ep), jnp.int32),
            jax.ShapeDtypeStruct((NC, np_pad), jnp.float32),
        ),
        mesh=_mesh(),
        scratch_types=(
            pltpu.VMEM((CH,), jnp.int32),
            pltpu.VMEM((CH,), jnp.int32),
            pltpu.VMEM((CH,), jnp.int32),
            pltpu.VMEM((CH,), jnp.float32),
            pltpu.VMEM((CH,), jnp.float32),
            pltpu.VMEM_SHARED((np_pad,), jnp.float32),
        ),
    )
    def prep(dst_hbm, dstl_hbm, degp_hbm, dbuf, l0b, l1b, ones, zb, dacc):
        c = lax.axis_index("c")
        s = lax.axis_index("s")
        wid = s * NC + c
        zv = jnp.zeros((LANES,), jnp.float32)
        ov = jnp.full((LANES,), 1.0, jnp.float32)
        for k in range(CH // LANES):
            zb[pl.ds(k * LANES, LANES)] = zv
            ones[pl.ds(k * LANES, LANES)] = ov

        @pl.loop(0, deg_stripe // CH)
        def _zero(i):
            pltpu.sync_copy(zb, dacc.at[pl.ds(s * deg_stripe + i * CH, CH)])

        plsc.subcore_barrier()

        @pl.loop(0, units)
        def _unit(u):
            off = (wid * units + u) * CH
            pltpu.sync_copy(dst_hbm.at[pl.ds(off, CH)], dbuf)
            for k in range(CH // LANES):
                v = dbuf[pl.ds(k * LANES, LANES)]
                tr = half + (v & (LANES - 1))
                l0b[pl.ds(k * LANES, LANES)] = jnp.where(v < half, v, tr)
                l1b[pl.ds(k * LANES, LANES)] = jnp.where(
                    v < half, tr, jnp.where(v < n_nodes, v - half, tr)
                )
            pltpu.sync_copy(l0b, dstl_hbm.at[0, pl.ds(off, CH)])
            pltpu.sync_copy(l1b, dstl_hbm.at[1, pl.ds(off, CH)])
            pltpu.sync_copy(ones, dacc.at[dbuf], add=True)

        plsc.subcore_barrier()
        pltpu.sync_copy(
            dacc.at[pl.ds(s * deg_stripe, deg_stripe)],
            degp_hbm.at[c, pl.ds(s * deg_stripe, deg_stripe)],
        )

    return prep


def _build_prop(ep, d, acc_pad):
    units = ep // (NS * CH)
    groups = units // NB
    stripe = acc_pad // NS

    scratch = (
        [pltpu.VMEM((CH,), jnp.int32) for _ in range(NB)]
        + [pltpu.VMEM((CH,), jnp.int32) for _ in range(NB)]
        + [pltpu.VMEM((CH, d), jnp.float32) for _ in range(NB)]
        + [pltpu.VMEM((CH, d), jnp.float32)]
        + [pltpu.VMEM_SHARED((acc_pad, d), jnp.float32)]
        + [pltpu.SemaphoreType.DMA for _ in range(3 * NB)]
    )

    @functools.partial(
        pl.kernel,
        out_type=jax.ShapeDtypeStruct((NC, acc_pad, d), jnp.float32),
        mesh=_mesh(),
        scratch_types=tuple(scratch),
    )
    def prop(y_hbm, src_hbm, dstl_hbm, out_hbm, *refs):
        sidx = refs[0:NB]
        didx = refs[NB : 2 * NB]
        rows = refs[2 * NB : 3 * NB]
        zb = refs[3 * NB]
        acc = refs[3 * NB + 1]
        sem_s = refs[3 * NB + 2 : 3 * NB + 2 + NB]
        sem_d = refs[3 * NB + 2 + NB : 3 * NB + 2 + 2 * NB]
        sem_g = refs[3 * NB + 2 + 2 * NB : 3 * NB + 2 + 3 * NB]
        c = lax.axis_index("c")
        s = lax.axis_index("s")
        zv = jnp.zeros((LANES,), jnp.float32)

        @pl.loop(0, CH)
        def _fill(r):
            for k in range(d // LANES):
                zb[r, pl.ds(k * LANES, LANES)] = zv

        @pl.loop(0, stripe // CH)
        def _zero(i):
            pltpu.sync_copy(zb, acc.at[pl.ds(s * stripe + i * CH, CH)])

        plsc.subcore_barrier()

        @pl.loop(0, groups)
        def _group(g):
            idx_descs = []
            for b in range(NB):
                off = (s * units + g * NB + b) * CH
                ds_ = pltpu.async_copy(src_hbm.at[pl.ds(off, CH)], sidx[b], sem_s[b])
                dd_ = pltpu.async_copy(
                    dstl_hbm.at[c, pl.ds(off, CH)], didx[b], sem_d[b]
                )
                idx_descs.append((ds_, dd_))
            gat = []
            for b in range(NB):
                idx_descs[b][0].wait()
                idx_descs[b][1].wait()
                gat.append(pltpu.async_copy(y_hbm.at[sidx[b]], rows[b], sem_g[b]))
            for b in range(NB):
                gat[b].wait()
                pltpu.sync_copy(rows[b], acc.at[didx[b]], add=True)

        plsc.subcore_barrier()
        pltpu.sync_copy(
            acc.at[pl.ds(s * stripe, stripe)],
            out_hbm.at[c, pl.ds(s * stripe, stripe)],
        )

    return prop


def _pre_body(g0_ref, g1_ref, dinv_ref):
    deg = g0_ref[...] + g1_ref[...]
    dinv_ref[...] = jnp.where(deg > 0, lax.rsqrt(deg), 0.0)


def _scale_y_body(x_ref, dm_ref, y_ref):
    y_ref[...] = x_ref[...] * dm_ref[...]


def _mid_body(s_ref, dm_ref, y_ref, p_ref):
    dm = dm_ref[...]
    p = s_ref[0] * dm
    p_ref[...] = p
    y_ref[...] = p * dm


def _post_body(s_ref, dm_ref, x0_ref, p1_ref, p2_ref, o_ref):
    o_ref[...] = 0.25 * (
        x0_ref[...] + p1_ref[...] + p2_ref[...] + s_ref[0] * dm_ref[...]
    )


def kernel(edge_index, emb_weight):
    n_nodes, d = emb_weight.shape
    half = n_nodes // 2
    e = edge_index.shape[1]
    ep = _rup(e, NC * NS * CH * NB)
    np_pad = _rup(n_nodes + 1, NS * CH)
    acc_pad = _rup(half + LANES, NS * CH)

    src = edge_index[0].astype(jnp.int32)
    dst = edge_index[1].astype(jnp.int32)
    pad = ep - e
    if pad:
        ar = jnp.arange(pad, dtype=jnp.int32)
        src = jnp.concatenate([src, ar % n_nodes])
        dst = jnp.concatenate([dst, n_nodes + ar % (np_pad - n_nodes)])

    prep = _build_prep(ep, n_nodes, half, np_pad)
    prop = _build_prop(ep, d, acc_pad)

    dstl, degp = prep(dst)

    r = 2000 if n_nodes % 2000 == 0 else n_nodes
    grid = (n_nodes // r,)
    br = half // r
    vec_spec = pl.BlockSpec((1, r), lambda i: (0, i))
    mat_spec = pl.BlockSpec((r, d), lambda i: (i, 0))
    s_spec = pl.BlockSpec((1, r, d), lambda i: (i // br, i % br, 0))

    degv = degp[:, :n_nodes]
    dinv = pl.pallas_call(
        _pre_body,
        grid=grid,
        in_specs=[
            pl.BlockSpec((1, r), lambda i: (0, i)),
            pl.BlockSpec((1, r), lambda i: (1, i)),
        ],
        out_specs=vec_spec,
        out_shape=jax.ShapeDtypeStruct((1, n_nodes), jnp.float32),
    )(degv, degv)

    dm = jnp.broadcast_to(dinv.reshape(n_nodes, 1), (n_nodes, d))

    x0 = emb_weight
    y = pl.pallas_call(
        _scale_y_body,
        grid=grid,
        in_specs=[mat_spec, mat_spec],
        out_specs=mat_spec,
        out_shape=jax.ShapeDtypeStruct((n_nodes, d), jnp.float32),
    )(x0, dm)

    ps = []
    out = None
    for layer in range(3):
        s_pad = prop(y, src, dstl)
        if layer < 2:
            y, p = pl.pallas_call(
                _mid_body,
                grid=grid,
                in_specs=[s_spec, mat_spec],
                out_specs=[mat_spec, mat_spec],
                out_shape=[
                    jax.ShapeDtypeStruct((n_nodes, d), jnp.float32),
                    jax.ShapeDtypeStruct((n_nodes, d), jnp.float32),
                ],
            )(s_pad, dm)
            ps.append(p)
        else:
            out = pl.pallas_call(
                _post_body,
                grid=grid,
                in_specs=[s_spec, mat_spec, mat_spec, mat_spec, mat_spec],
                out_specs=mat_spec,
                out_shape=jax.ShapeDtypeStruct((n_nodes, d), jnp.float32),
            )(s_pad, dm, x0, ps[0], ps[1])

    return out[:half], out[half:]


# speedup vs baseline: 20.3087x; 20.3087x over previous
import functools

import jax
import jax.numpy as jnp
from jax import lax
from jax.experimental import pallas as pl
from jax.experimental.pallas import tpu as pltpu
from jax.experimental.pallas import tpu_sc as plsc

NC = 2
NS = 16
LANES = 16
CH = 128
NB = 4
ZR = 64


def _rup(x, m):
    return (x + m - 1) // m * m


def _mesh():
    return plsc.VectorSubcoreMesh(
        core_axis_name="c", subcore_axis_name="s", num_cores=NC, num_subcores=NS
    )


def _build_prep(ep, n_nodes, half, np_pad):
    units = ep // (NC * NS * CH)
    deg_stripe = np_pad // NS

    @functools.partial(
        pl.kernel,
        out_type=(
            jax.ShapeDtypeStruct((NC, ep), jnp.int32),
            jax.ShapeDtypeStruct((NC, np_pad), jnp.float32),
        ),
        mesh=_mesh(),
        compiler_params=pltpu.CompilerParams(use_tc_tiling_on_sc=False),
        scratch_types=(
            pltpu.VMEM((CH,), jnp.int32),
            pltpu.VMEM((CH,), jnp.int32),
            pltpu.VMEM((CH,), jnp.int32),
            pltpu.VMEM((CH,), jnp.float32),
            pltpu.VMEM((CH,), jnp.float32),
            pltpu.VMEM_SHARED((np_pad,), jnp.float32),
        ),
    )
    def prep(dst_hbm, dstl_hbm, degp_hbm, dbuf, l0b, l1b, ones, zb, dacc):
        c = lax.axis_index("c")
        s = lax.axis_index("s")
        wid = s * NC + c
        zv = jnp.zeros((LANES,), jnp.float32)
        ov = jnp.full((LANES,), 1.0, jnp.float32)
        for k in range(CH // LANES):
            zb[pl.ds(k * LANES, LANES)] = zv
            ones[pl.ds(k * LANES, LANES)] = ov

        @pl.loop(0, deg_stripe // CH)
        def _zero(i):
            pltpu.sync_copy(zb, dacc.at[pl.ds(s * deg_stripe + i * CH, CH)])

        plsc.subcore_barrier()

        @pl.loop(0, units)
        def _unit(u):
            off = (wid * units + u) * CH
            pltpu.sync_copy(dst_hbm.at[pl.ds(off, CH)], dbuf)
            for k in range(CH // LANES):
                v = dbuf[pl.ds(k * LANES, LANES)]
                tr = half + (v & (LANES - 1))
                l0b[pl.ds(k * LANES, LANES)] = jnp.where(v < half, v, tr)
                l1b[pl.ds(k * LANES, LANES)] = jnp.where(
                    v < half, tr, jnp.where(v < n_nodes, v - half, tr)
                )
            pltpu.sync_copy(l0b, dstl_hbm.at[0, pl.ds(off, CH)])
            pltpu.sync_copy(l1b, dstl_hbm.at[1, pl.ds(off, CH)])
            pltpu.sync_copy(ones, dacc.at[dbuf], add=True)

        plsc.subcore_barrier()
        pltpu.sync_copy(
            dacc.at[pl.ds(s * deg_stripe, deg_stripe)],
            degp_hbm.at[c, pl.ds(s * deg_stripe, deg_stripe)],
        )

    return prep


def _build_prop(ep, d, acc_pad):
    units = ep // (NS * CH)
    groups = units // NB
    stripe = acc_pad // NS

    scratch = (
        [pltpu.VMEM((CH,), jnp.int32) for _ in range(NB)]
        + [pltpu.VMEM((CH,), jnp.int32) for _ in range(NB)]
        + [pltpu.VMEM((CH, d), jnp.float32) for _ in range(NB)]
        + [pltpu.VMEM_SHARED((acc_pad, d), jnp.float32)]
        + [pltpu.SemaphoreType.DMA for _ in range(3 * NB)]
    )

    @functools.partial(
        pl.kernel,
        out_type=jax.ShapeDtypeStruct((NC, acc_pad, d), jnp.float32),
        mesh=_mesh(),
        compiler_params=pltpu.CompilerParams(use_tc_tiling_on_sc=False),
        scratch_types=tuple(scratch),
    )
    def prop(y_hbm, src_hbm, dstl_hbm, out_hbm, *refs):
        sidx = refs[0:NB]
        didx = refs[NB : 2 * NB]
        rows = refs[2 * NB : 3 * NB]
        acc = refs[3 * NB]
        sem_s = refs[3 * NB + 1 : 3 * NB + 1 + NB]
        sem_d = refs[3 * NB + 1 + NB : 3 * NB + 1 + 2 * NB]
        sem_g = refs[3 * NB + 1 + 2 * NB : 3 * NB + 1 + 3 * NB]
        c = lax.axis_index("c")
        s = lax.axis_index("s")
        zv = jnp.zeros((LANES,), jnp.float32)

        @pl.loop(0, ZR)
        def _fill(r):
            for k in range(d // LANES):
                rows[0][r, pl.ds(k * LANES, LANES)] = zv

        @pl.loop(0, stripe // ZR)
        def _zero(i):
            pltpu.sync_copy(
                rows[0].at[pl.ds(0, ZR)], acc.at[pl.ds(s * stripe + i * ZR, ZR)]
            )

        plsc.subcore_barrier()

        @pl.loop(0, groups)
        def _group(g):
            idx_descs = []
            for b in range(NB):
                off = (s * units + g * NB + b) * CH
                ds_ = pltpu.async_copy(src_hbm.at[pl.ds(off, CH)], sidx[b], sem_s[b])
                dd_ = pltpu.async_copy(
                    dstl_hbm.at[c, pl.ds(off, CH)], didx[b], sem_d[b]
                )
                idx_descs.append((ds_, dd_))
            gat = []
            for b in range(NB):
                idx_descs[b][0].wait()
                idx_descs[b][1].wait()
                gat.append(pltpu.async_copy(y_hbm.at[sidx[b]], rows[b], sem_g[b]))
            for b in range(NB):
                gat[b].wait()
                pltpu.sync_copy(rows[b], acc.at[didx[b]], add=True)

        plsc.subcore_barrier()
        pltpu.sync_copy(
            acc.at[pl.ds(s * stripe, stripe)],
            out_hbm.at[c, pl.ds(s * stripe, stripe)],
        )

    return prop


def _pre_body(g0_ref, g1_ref, dinv_ref):
    deg = g0_ref[...] + g1_ref[...]
    dinv_ref[...] = jnp.where(deg > 0, lax.rsqrt(deg), 0.0)


def _scale_y_body(x_ref, dm_ref, y_ref):
    y_ref[...] = x_ref[...] * dm_ref[...]


def _mid_body(s_ref, dm_ref, y_ref, p_ref):
    dm = dm_ref[...]
    p = s_ref[0] * dm
    p_ref[...] = p
    y_ref[...] = p * dm


def _post_body(s_ref, dm_ref, x0_ref, p1_ref, p2_ref, o_ref):
    o_ref[...] = 0.25 * (
        x0_ref[...] + p1_ref[...] + p2_ref[...] + s_ref[0] * dm_ref[...]
    )


def kernel(edge_index, emb_weight):
    n_nodes, d = emb_weight.shape
    half = n_nodes // 2
    e = edge_index.shape[1]
    ep = _rup(e, NC * NS * CH * NB)
    np_pad = _rup(n_nodes + 1, NS * CH)
    acc_pad = _rup(half + LANES, NS * ZR)

    src = edge_index[0].astype(jnp.int32)
    dst = edge_index[1].astype(jnp.int32)
    pad = ep - e
    if pad:
        ar = jnp.arange(pad, dtype=jnp.int32)
        src = jnp.concatenate([src, ar % n_nodes])
        dst = jnp.concatenate([dst, n_nodes + ar % (np_pad - n_nodes)])

    prep = _build_prep(ep, n_nodes, half, np_pad)
    prop = _build_prop(ep, d, acc_pad)

    dstl, degp = prep(dst)

    r = 2000 if n_nodes % 2000 == 0 else n_nodes
    grid = (n_nodes // r,)
    br = half // r
    mat_spec = pl.BlockSpec((r, d), lambda i: (i, 0))
    s_spec = pl.BlockSpec((1, r, d), lambda i: (i // br, i % br, 0))

    whole_spec = pl.BlockSpec((1, n_nodes), lambda: (0, 0))
    dinv = pl.pallas_call(
        _pre_body,
        grid=(),
        in_specs=[whole_spec, whole_spec],
        out_specs=whole_spec,
        out_shape=jax.ShapeDtypeStruct((1, n_nodes), jnp.float32),
    )(degp[0:1, :n_nodes], degp[1:2, :n_nodes])

    dm = jnp.broadcast_to(dinv.reshape(n_nodes, 1), (n_nodes, d))

    x0 = emb_weight
    y = pl.pallas_call(
        _scale_y_body,
        grid=grid,
        in_specs=[mat_spec, mat_spec],
        out_specs=mat_spec,
        out_shape=jax.ShapeDtypeStruct((n_nodes, d), jnp.float32),
    )(x0, dm)

    ps = []
    out = None
    for layer in range(3):
        s_pad = prop(y, src, dstl)
        if layer < 2:
            y, p = pl.pallas_call(
                _mid_body,
                grid=grid,
                in_specs=[s_spec, mat_spec],
                out_specs=[mat_spec, mat_spec],
                out_shape=[
                    jax.ShapeDtypeStruct((n_nodes, d), jnp.float32),
                    jax.ShapeDtypeStruct((n_nodes, d), jnp.float32),
                ],
            )(s_pad, dm)
            ps.append(p)
        else:
            out = pl.pallas_call(
                _post_body,
                grid=grid,
                in_specs=[s_spec, mat_spec, mat_spec, mat_spec, mat_spec],
                out_specs=mat_spec,
                out_shape=jax.ShapeDtypeStruct((n_nodes, d), jnp.float32),
            )(s_pad, dm, x0, ps[0], ps[1])

    return out[:half], out[half:]


# speedup vs baseline: 21.6228x; 1.0647x over previous
import functools

import jax
import jax.numpy as jnp
from jax import lax
from jax.experimental import pallas as pl
from jax.experimental.pallas import tpu as pltpu
from jax.experimental.pallas import tpu_sc as plsc

NC = 2
NS = 16
LANES = 16
CH = 128
NB = 8


def _rup(x, m):
    return (x + m - 1) // m * m


def _mesh():
    return plsc.VectorSubcoreMesh(
        core_axis_name="c", subcore_axis_name="s", num_cores=NC, num_subcores=NS
    )


_SC_PARAMS = pltpu.CompilerParams(use_tc_tiling_on_sc=False)


def _build_prep(ep, np_pad):
    units = ep // (NC * NS * CH)
    deg_stripe = np_pad // NS

    @functools.partial(
        pl.kernel,
        out_type=jax.ShapeDtypeStruct((NC, np_pad), jnp.float32),
        mesh=_mesh(),
        compiler_params=_SC_PARAMS,
        scratch_types=(
            pltpu.VMEM((CH,), jnp.int32),
            pltpu.VMEM((CH,), jnp.float32),
            pltpu.VMEM((CH,), jnp.float32),
            pltpu.VMEM_SHARED((np_pad,), jnp.float32),
        ),
    )
    def prep(dst_hbm, degp_hbm, dbuf, ones, zb, dacc):
        c = lax.axis_index("c")
        s = lax.axis_index("s")
        wid = s * NC + c
        zv = jnp.zeros((LANES,), jnp.float32)
        ov = jnp.full((LANES,), 1.0, jnp.float32)
        for k in range(CH // LANES):
            zb[pl.ds(k * LANES, LANES)] = zv
            ones[pl.ds(k * LANES, LANES)] = ov

        @pl.loop(0, deg_stripe // CH)
        def _zero(i):
            pltpu.sync_copy(zb, dacc.at[pl.ds(s * deg_stripe + i * CH, CH)])

        plsc.subcore_barrier()

        @pl.loop(0, units)
        def _unit(u):
            off = (wid * units + u) * CH
            pltpu.sync_copy(dst_hbm.at[pl.ds(off, CH)], dbuf)
            pltpu.sync_copy(ones, dacc.at[dbuf], add=True)

        plsc.subcore_barrier()
        pltpu.sync_copy(
            dacc.at[pl.ds(s * deg_stripe, deg_stripe)],
            degp_hbm.at[c, pl.ds(s * deg_stripe, deg_stripe)],
        )

    return prep


def _build_prop(ep, np_pad, hd):
    units = ep // (NS * CH)
    groups = units // NB
    stripe = np_pad // NS

    @functools.partial(
        pl.kernel,
        out_type=jax.ShapeDtypeStruct((NC, np_pad, hd), jnp.float32),
        mesh=_mesh(),
        compiler_params=_SC_PARAMS,
        scratch_types=tuple(
            [pltpu.VMEM((CH,), jnp.int32) for _ in range(NB)]
            + [pltpu.VMEM((CH,), jnp.int32) for _ in range(NB)]
            + [pltpu.VMEM((CH, hd), jnp.float32) for _ in range(NB)]
            + [pltpu.VMEM_SHARED((np_pad, hd), jnp.float32)]
            + [pltpu.SemaphoreType.DMA for _ in range(3 * NB)]
        ),
    )
    def prop(z_hbm, src_hbm, dst_hbm, out_hbm, *refs):
        sidx = refs[0:NB]
        didx = refs[NB : 2 * NB]
        rows = refs[2 * NB : 3 * NB]
        acc = refs[3 * NB]
        base = 3 * NB + 1
        sem_s = refs[base : base + NB]
        sem_d = refs[base + NB : base + 2 * NB]
        sem_g = refs[base + 2 * NB : base + 3 * NB]
        c = lax.axis_index("c")
        s = lax.axis_index("s")
        zrows = z_hbm.shape[0] // NC
        rebase = c * zrows
        zv = jnp.zeros((LANES,), jnp.float32)

        @pl.loop(0, CH)
        def _fill(r):
            for k in range(hd // LANES):
                rows[0][r, pl.ds(k * LANES, LANES)] = zv

        @pl.loop(0, stripe // CH)
        def _zero(i):
            pltpu.sync_copy(rows[0], acc.at[pl.ds(s * stripe + i * CH, CH)])

        plsc.subcore_barrier()

        @pl.loop(0, groups)
        def _group(g):
            descs = []
            for b in range(NB):
                off = (s * units + g * NB + b) * CH
                d1 = pltpu.async_copy(src_hbm.at[pl.ds(off, CH)], sidx[b], sem_s[b])
                d2 = pltpu.async_copy(dst_hbm.at[pl.ds(off, CH)], didx[b], sem_d[b])
                descs.append((d1, d2))
            gat = []
            for b in range(NB):
                descs[b][0].wait()
                for k in range(CH // LANES):
                    sl = pl.ds(k * LANES, LANES)
                    sidx[b][sl] = sidx[b][sl] + rebase
                gat.append(pltpu.async_copy(z_hbm.at[sidx[b]], rows[b], sem_g[b]))
            for b in range(NB):
                gat[b].wait()
                descs[b][1].wait()
                pltpu.sync_copy(rows[b], acc.at[didx[b]], add=True)

        plsc.subcore_barrier()
        pltpu.sync_copy(
            acc.at[pl.ds(s * stripe, stripe)],
            out_hbm.at[c, pl.ds(s * stripe, stripe)],
        )

    return prop


def _pre_body(g0_ref, g1_ref, dinv_ref):
    deg = g0_ref[...] + g1_ref[...]
    dinv_ref[...] = jnp.where(deg > 0, lax.rsqrt(deg), 0.0)


def _split_body(x_ref, dm_ref, za_ref, zb_ref, *, hd):
    z = x_ref[...] * dm_ref[...]
    za_ref[...] = z[:, :hd]
    zb_ref[...] = z[:, hd:]


def _mid_body(sa_ref, sb_ref, dm_ref, za_ref, zb_ref, p_ref, *, hd):
    dm = dm_ref[...]
    sc = jnp.concatenate([sa_ref[0], sb_ref[0]], axis=1)
    p = sc * dm
    p_ref[...] = p
    y = p * dm
    za_ref[...] = y[:, :hd]
    zb_ref[...] = y[:, hd:]


def _post_body(sa_ref, sb_ref, dm_ref, x0_ref, p1_ref, p2_ref, o_ref):
    sc = jnp.concatenate([sa_ref[0], sb_ref[0]], axis=1)
    o_ref[...] = 0.25 * (
        x0_ref[...] + p1_ref[...] + p2_ref[...] + sc * dm_ref[...]
    )


def kernel(edge_index, emb_weight):
    n_nodes, d = emb_weight.shape
    hd = d // 2
    half = n_nodes // 2
    e = edge_index.shape[1]
    ep = _rup(e, NS * CH * NB)
    np_pad = _rup(n_nodes + 1, NS * CH)

    src = edge_index[0].astype(jnp.int32)
    dst = edge_index[1].astype(jnp.int32)
    pad = ep - e
    if pad:
        ar = jnp.arange(pad, dtype=jnp.int32)
        src = jnp.concatenate([src, ar % n_nodes])
        dst = jnp.concatenate([dst, n_nodes + ar % (np_pad - n_nodes)])

    degp = _build_prep(ep, np_pad)(dst)

    r = 2000 if n_nodes % 2000 == 0 else n_nodes
    grid = (n_nodes // r,)
    vec_spec = pl.BlockSpec((1, n_nodes), lambda: (0, 0))
    mat_spec = pl.BlockSpec((r, d), lambda i: (i, 0))
    hmat_spec = pl.BlockSpec((r, hd), lambda i: (i, 0))
    sa_spec = pl.BlockSpec((1, r, hd), lambda i: (0, i, 0))
    sb_spec = pl.BlockSpec((1, r, hd), lambda i: (1, i, 0))

    dinv = pl.pallas_call(
        _pre_body,
        grid=(),
        in_specs=[vec_spec, vec_spec],
        out_specs=vec_spec,
        out_shape=jax.ShapeDtypeStruct((1, n_nodes), jnp.float32),
    )(degp[0:1, :n_nodes], degp[1:2, :n_nodes])

    dm = jnp.broadcast_to(dinv.reshape(n_nodes, 1), (n_nodes, d))

    x0 = emb_weight
    z0a, z0b = pl.pallas_call(
        functools.partial(_split_body, hd=hd),
        grid=grid,
        in_specs=[mat_spec, mat_spec],
        out_specs=[hmat_spec, hmat_spec],
        out_shape=[jax.ShapeDtypeStruct((n_nodes, hd), jnp.float32)] * 2,
    )(x0, dm)
    z = jnp.stack([z0a, z0b]).reshape(NC * n_nodes, hd)

    prop = _build_prop(ep, np_pad, hd)
    ps = []
    out = None
    for layer in range(3):
        s_pad = prop(z, src, dst)
        if layer < 2:
            za, zb, p = pl.pallas_call(
                functools.partial(_mid_body, hd=hd),
                grid=grid,
                in_specs=[sa_spec, sb_spec, mat_spec],
                out_specs=[hmat_spec, hmat_spec, mat_spec],
                out_shape=[
                    jax.ShapeDtypeStruct((n_nodes, hd), jnp.float32),
                    jax.ShapeDtypeStruct((n_nodes, hd), jnp.float32),
                    jax.ShapeDtypeStruct((n_nodes, d), jnp.float32),
                ],
            )(s_pad, s_pad, dm)
            ps.append(p)
            z = jnp.stack([za, zb]).reshape(NC * n_nodes, hd)
        else:
            out = pl.pallas_call(
                _post_body,
                grid=grid,
                in_specs=[sa_spec, sb_spec, mat_spec, mat_spec, mat_spec,
                          mat_spec],
                out_specs=mat_spec,
                out_shape=jax.ShapeDtypeStruct((n_nodes, d), jnp.float32),
            )(s_pad, s_pad, dm, x0, ps[0], ps[1])

    return out[:half], out[half:]
